# initial kernel scaffold (unmeasured)
import jax
import jax.numpy as jnp
from jax import lax
from jax.experimental import pallas as pl
from jax.experimental.pallas import tpu as pltpu

N_DEV = 32
N_RIGHT = 16
N_LEFT = 15


def kernel(x, w_mat):
    m_per, k = x.shape
    _, n_per = w_mat.shape

    def body(x_ref, w_ref, out_ref, comm_ref, send_r, recv_r, send_l, recv_l):
        my = lax.axis_index("i")
        left = lax.rem(my + N_DEV - 1, N_DEV)
        right = lax.rem(my + 1, N_DEV)

        barrier = pltpu.get_barrier_semaphore()
        for nbr in (left, right):
            pl.semaphore_signal(
                barrier, inc=1, device_id=(nbr,),
                device_id_type=pl.DeviceIdType.MESH,
            )
        pl.semaphore_wait(barrier, 2)

        def gemm_chunk(chunk, origin):
            y = jnp.dot(chunk, w_ref[...], preferred_element_type=jnp.float32)
            out_ref[pl.ds(origin * m_per, m_per), :] = jnp.maximum(y, 0.0)

        gemm_chunk(x_ref[...], my)

        for h in range(N_RIGHT):
            idx_r = lax.rem(my + N_DEV - h, N_DEV)
            rdma_r = pltpu.make_async_remote_copy(
                src_ref=x_ref if h == 0 else comm_ref.at[idx_r],
                dst_ref=comm_ref.at[idx_r],
                send_sem=send_r.at[h],
                recv_sem=recv_r.at[h],
                device_id=(right,),
                device_id_type=pl.DeviceIdType.MESH,
            )
            rdma_r.start()

            if h < N_LEFT:
                idx_l = lax.rem(my + h, N_DEV)
                rdma_l = pltpu.make_async_remote_copy(
                    src_ref=x_ref if h == 0 else comm_ref.at[idx_l],
                    dst_ref=comm_ref.at[idx_l],
                    send_sem=send_l.at[h],
                    recv_sem=recv_l.at[h],
                    device_id=(left,),
                    device_id_type=pl.DeviceIdType.MESH,
                )
                rdma_l.start()

            rdma_r.wait()
            origin_r = lax.rem(my + N_DEV - h - 1, N_DEV)
            gemm_chunk(comm_ref[origin_r], origin_r)

            if h < N_LEFT:
                rdma_l.wait()
                origin_l = lax.rem(my + h + 1, N_DEV)
                gemm_chunk(comm_ref[origin_l], origin_l)

    return pl.pallas_call(
        body,
        out_shape=jax.ShapeDtypeStruct((N_DEV * m_per, n_per), jnp.float32),
        in_specs=[
            pl.BlockSpec(memory_space=pltpu.VMEM),
            pl.BlockSpec(memory_space=pltpu.VMEM),
        ],
        out_specs=pl.BlockSpec(memory_space=pltpu.VMEM),
        scratch_shapes=[
            pltpu.VMEM((N_DEV, m_per, k), jnp.float32),
            pltpu.SemaphoreType.DMA((N_RIGHT,)),
            pltpu.SemaphoreType.DMA((N_RIGHT,)),
            pltpu.SemaphoreType.DMA((N_LEFT,)),
            pltpu.SemaphoreType.DMA((N_LEFT,)),
        ],
        compiler_params=pltpu.CompilerParams(collective_id=0),
    )(x, w_mat)


# baseline (device time: 762463 ns/iter reference)
import jax
import jax.numpy as jnp
from jax import lax
from jax.experimental import pallas as pl
from jax.experimental.pallas import tpu as pltpu

N_DEV = 32
N_RIGHT = 16
N_LEFT = 15
S = 4


def kernel(x, w_mat):
    m_per, k = x.shape
    _, n_per = w_mat.shape

    def body(x_ref, w_ref, out_ref, comm_r, comm_l,
             send_r, recv_r, send_l, recv_l, credit_r, credit_l):
        my = lax.axis_index("i")
        left = lax.rem(my + N_DEV - 1, N_DEV)
        right = lax.rem(my + 1, N_DEV)

        barrier = pltpu.get_barrier_semaphore()
        for nbr in (left, right):
            pl.semaphore_signal(
                barrier, inc=1, device_id=(nbr,),
                device_id_type=pl.DeviceIdType.MESH,
            )
        pl.semaphore_wait(barrier, 2)

        def gemm_chunk(chunk, origin):
            y = jnp.dot(chunk, w_ref[...], preferred_element_type=jnp.float32)
            out_ref[pl.ds(origin * m_per, m_per), :] = jnp.maximum(y, 0.0)

        gemm_chunk(x_ref[...], my)

        for h in range(N_RIGHT):
            s = h % S
            if h >= S:
                pl.semaphore_wait(credit_r, 1)
            rdma_r = pltpu.make_async_remote_copy(
                src_ref=x_ref if h == 0 else comm_r.at[(h - 1) % S],
                dst_ref=comm_r.at[s],
                send_sem=send_r.at[s],
                recv_sem=recv_r.at[s],
                device_id=(right,),
                device_id_type=pl.DeviceIdType.MESH,
            )
            rdma_r.start()

            if h < N_LEFT:
                if h >= S:
                    pl.semaphore_wait(credit_l, 1)
                rdma_l = pltpu.make_async_remote_copy(
                    src_ref=x_ref if h == 0 else comm_l.at[(h - 1) % S],
                    dst_ref=comm_l.at[s],
                    send_sem=send_l.at[s],
                    recv_sem=recv_l.at[s],
                    device_id=(left,),
                    device_id_type=pl.DeviceIdType.MESH,
                )
                rdma_l.start()

            rdma_r.wait()
            gemm_chunk(comm_r[s], lax.rem(my + N_DEV - h - 1, N_DEV))
            if 1 <= h <= N_RIGHT - S:
                pl.semaphore_signal(
                    credit_r, inc=1, device_id=(left,),
                    device_id_type=pl.DeviceIdType.MESH,
                )

            if h < N_LEFT:
                rdma_l.wait()
                gemm_chunk(comm_l[s], lax.rem(my + h + 1, N_DEV))
                if 1 <= h <= N_LEFT - S:
                    pl.semaphore_signal(
                        credit_l, inc=1, device_id=(right,),
                        device_id_type=pl.DeviceIdType.MESH,
                    )

    return pl.pallas_call(
        body,
        out_shape=jax.ShapeDtypeStruct((N_DEV * m_per, n_per), jnp.float32),
        in_specs=[
            pl.BlockSpec(memory_space=pltpu.VMEM),
            pl.BlockSpec(memory_space=pltpu.VMEM),
        ],
        out_specs=pl.BlockSpec(memory_space=pltpu.VMEM),
        scratch_shapes=[
            pltpu.VMEM((S, m_per, k), jnp.float32),
            pltpu.VMEM((S, m_per, k), jnp.float32),
            pltpu.SemaphoreType.DMA((S,)),
            pltpu.SemaphoreType.DMA((S,)),
            pltpu.SemaphoreType.DMA((S,)),
            pltpu.SemaphoreType.DMA((S,)),
            pltpu.SemaphoreType.REGULAR,
            pltpu.SemaphoreType.REGULAR,
        ],
        compiler_params=pltpu.CompilerParams(collective_id=0),
    )(x, w_mat)


# device time: 417749 ns/iter; 1.8252x vs baseline; 1.8252x over previous
import jax
import jax.numpy as jnp
from jax import lax
from jax.experimental import pallas as pl
from jax.experimental.pallas import tpu as pltpu

N_DEV = 32
N_RIGHT = 16
N_LEFT = 15
S = 4

RING = [0, 8, 16, 24, 27, 28, 31, 30, 29, 26, 25, 17, 18, 21, 22, 23,
        20, 19, 11, 12, 15, 14, 13, 10, 9, 1, 2, 5, 6, 7, 4, 3]


def kernel(x, w_mat):
    m_per, k = x.shape
    _, n_per = w_mat.shape

    my = lax.axis_index("i")
    ring = jnp.array(RING, dtype=jnp.int32)
    pos = jnp.argmax(ring == my).astype(jnp.int32)
    origins_r = ring[jnp.mod(pos - 1 - jnp.arange(N_RIGHT, dtype=jnp.int32), N_DEV)]
    origins_l = ring[jnp.mod(pos + 1 + jnp.arange(N_LEFT, dtype=jnp.int32), N_DEV)]
    meta = jnp.concatenate([
        ring[jnp.mod(pos - 1, N_DEV)][None],
        ring[jnp.mod(pos + 1, N_DEV)][None],
        origins_r,
        origins_l,
    ]).astype(jnp.int32)

    def body(meta_ref, x_ref, w_ref, out_ref, comm_r, comm_l,
             send_r, recv_r, send_l, recv_l, credit_r, credit_l):
        my_dev = lax.axis_index("i")
        left = meta_ref[0]
        right = meta_ref[1]

        barrier = pltpu.get_barrier_semaphore()
        for nbr in (left, right):
            pl.semaphore_signal(
                barrier, inc=1, device_id=(nbr,),
                device_id_type=pl.DeviceIdType.MESH,
            )
        pl.semaphore_wait(barrier, 2)

        def gemm_chunk(chunk, origin):
            y = jnp.dot(chunk, w_ref[...], preferred_element_type=jnp.float32)
            out_ref[pl.ds(origin * m_per, m_per), :] = jnp.maximum(y, 0.0)

        gemm_chunk(x_ref[...], my_dev)

        for h in range(N_RIGHT):
            s = h % S
            if h >= S:
                pl.semaphore_wait(credit_r, 1)
            rdma_r = pltpu.make_async_remote_copy(
                src_ref=x_ref if h == 0 else comm_r.at[(h - 1) % S],
                dst_ref=comm_r.at[s],
                send_sem=send_r.at[s],
                recv_sem=recv_r.at[s],
                device_id=(right,),
                device_id_type=pl.DeviceIdType.MESH,
            )
            rdma_r.start()

            if h < N_LEFT:
                if h >= S:
                    pl.semaphore_wait(credit_l, 1)
                rdma_l = pltpu.make_async_remote_copy(
                    src_ref=x_ref if h == 0 else comm_l.at[(h - 1) % S],
                    dst_ref=comm_l.at[s],
                    send_sem=send_l.at[s],
                    recv_sem=recv_l.at[s],
                    device_id=(left,),
                    device_id_type=pl.DeviceIdType.MESH,
                )
                rdma_l.start()

            rdma_r.wait()
            gemm_chunk(comm_r[s], meta_ref[2 + h])
            if 1 <= h <= N_RIGHT - S:
                pl.semaphore_signal(
                    credit_r, inc=1, device_id=(left,),
                    device_id_type=pl.DeviceIdType.MESH,
                )

            if h < N_LEFT:
                rdma_l.wait()
                gemm_chunk(comm_l[s], meta_ref[18 + h])
                if 1 <= h <= N_LEFT - S:
                    pl.semaphore_signal(
                        credit_l, inc=1, device_id=(right,),
                        device_id_type=pl.DeviceIdType.MESH,
                    )

    return pl.pallas_call(
        body,
        out_shape=jax.ShapeDtypeStruct((N_DEV * m_per, n_per), jnp.float32),
        in_specs=[
            pl.BlockSpec(memory_space=pltpu.SMEM),
            pl.BlockSpec(memory_space=pltpu.VMEM),
            pl.BlockSpec(memory_space=pltpu.VMEM),
        ],
        out_specs=pl.BlockSpec(memory_space=pltpu.VMEM),
        scratch_shapes=[
            pltpu.VMEM((S, m_per, k), jnp.float32),
            pltpu.VMEM((S, m_per, k), jnp.float32),
            pltpu.SemaphoreType.DMA((S,)),
            pltpu.SemaphoreType.DMA((S,)),
            pltpu.SemaphoreType.DMA((S,)),
            pltpu.SemaphoreType.DMA((S,)),
            pltpu.SemaphoreType.REGULAR,
            pltpu.SemaphoreType.REGULAR,
        ],
        compiler_params=pltpu.CompilerParams(collective_id=0),
    )(meta, x, w_mat)


# device time: 402807 ns/iter; 1.8929x vs baseline; 1.0371x over previous
import jax
import jax.numpy as jnp
from jax import lax
from jax.experimental import pallas as pl
from jax.experimental.pallas import tpu as pltpu

N_DEV = 32
N_RIGHT = 16
N_LEFT = 15
S = 4

RING = [0, 8, 16, 24, 27, 28, 31, 30, 29, 26, 25, 17, 18, 21, 22, 23,
        20, 19, 11, 12, 15, 14, 13, 10, 9, 1, 2, 5, 6, 7, 4, 3]


def kernel(x, w_mat):
    m_per, k = x.shape
    _, n_per = w_mat.shape

    my = lax.axis_index("i")
    ring = jnp.array(RING, dtype=jnp.int32)
    pos = jnp.argmax(ring == my).astype(jnp.int32)
    origins_r = ring[jnp.mod(pos - 1 - jnp.arange(N_RIGHT, dtype=jnp.int32), N_DEV)]
    origins_l = ring[jnp.mod(pos + 1 + jnp.arange(N_LEFT, dtype=jnp.int32), N_DEV)]
    meta = jnp.concatenate([
        ring[jnp.mod(pos - 1, N_DEV)][None],
        ring[jnp.mod(pos + 1, N_DEV)][None],
        origins_r,
        origins_l,
    ]).astype(jnp.int32)

    def body(meta_ref, x_ref, w_ref, out_ref, comm_r, comm_l,
             send_r, recv_r, send_l, recv_l, credit_r, credit_l):
        my_dev = lax.axis_index("i")
        left = meta_ref[0]
        right = meta_ref[1]

        barrier = pltpu.get_barrier_semaphore()
        for nbr in (left, right):
            pl.semaphore_signal(
                barrier, inc=1, device_id=(nbr,),
                device_id_type=pl.DeviceIdType.MESH,
            )
        pl.semaphore_wait(barrier, 2)

        def gemm_chunk(chunk, origin):
            y = jnp.dot(chunk, w_ref[...], preferred_element_type=jnp.float32)
            out_ref[pl.ds(origin * m_per, m_per), :] = jnp.maximum(y, 0.0)

        prev_r = None
        prev_l = None
        for h in range(N_RIGHT):
            s = h % S
            if h >= S:
                pl.semaphore_wait(credit_r, 1)
            if h >= S and h < N_LEFT:
                pl.semaphore_wait(credit_l, 1)

            rdma_r = pltpu.make_async_remote_copy(
                src_ref=x_ref if h == 0 else comm_r.at[(h - 1) % S],
                dst_ref=comm_r.at[s],
                send_sem=send_r.at[s],
                recv_sem=recv_r.at[s],
                device_id=(right,),
                device_id_type=pl.DeviceIdType.MESH,
            )
            rdma_r.start()
            rdma_l = None
            if h < N_LEFT:
                rdma_l = pltpu.make_async_remote_copy(
                    src_ref=x_ref if h == 0 else comm_l.at[(h - 1) % S],
                    dst_ref=comm_l.at[s],
                    send_sem=send_l.at[s],
                    recv_sem=recv_l.at[s],
                    device_id=(left,),
                    device_id_type=pl.DeviceIdType.MESH,
                )
                rdma_l.start()

            if prev_r is not None:
                prev_r.wait_send()
                if 0 <= h - 2 <= N_RIGHT - S - 1:
                    pl.semaphore_signal(
                        credit_r, inc=1, device_id=(left,),
                        device_id_type=pl.DeviceIdType.MESH,
                    )
            if prev_l is not None:
                prev_l.wait_send()
                if 0 <= h - 2 <= N_LEFT - S - 1:
                    pl.semaphore_signal(
                        credit_l, inc=1, device_id=(right,),
                        device_id_type=pl.DeviceIdType.MESH,
                    )

            if h == 0:
                gemm_chunk(x_ref[...], my_dev)
            else:
                gemm_chunk(comm_r[(h - 1) % S], meta_ref[2 + (h - 1)])
                gemm_chunk(comm_l[(h - 1) % S], meta_ref[18 + (h - 1)])

            rdma_r.wait_recv()
            if rdma_l is not None:
                rdma_l.wait_recv()

            prev_r = rdma_r
            if rdma_l is not None:
                prev_l = rdma_l

        prev_r.wait_send()
        gemm_chunk(comm_r[(N_RIGHT - 1) % S], meta_ref[2 + (N_RIGHT - 1)])

    return pl.pallas_call(
        body,
        out_shape=jax.ShapeDtypeStruct((N_DEV * m_per, n_per), jnp.float32),
        in_specs=[
            pl.BlockSpec(memory_space=pltpu.SMEM),
            pl.BlockSpec(memory_space=pltpu.VMEM),
            pl.BlockSpec(memory_space=pltpu.VMEM),
        ],
        out_specs=pl.BlockSpec(memory_space=pltpu.VMEM),
        scratch_shapes=[
            pltpu.VMEM((S, m_per, k), jnp.float32),
            pltpu.VMEM((S, m_per, k), jnp.float32),
            pltpu.SemaphoreType.DMA((S,)),
            pltpu.SemaphoreType.DMA((S,)),
            pltpu.SemaphoreType.DMA((S,)),
            pltpu.SemaphoreType.DMA((S,)),
            pltpu.SemaphoreType.REGULAR,
            pltpu.SemaphoreType.REGULAR,
        ],
        compiler_params=pltpu.CompilerParams(collective_id=0),
    )(meta, x, w_mat)


# device time: 374775 ns/iter; 2.0345x vs baseline; 1.0748x over previous
import jax
import jax.numpy as jnp
from jax import lax
from jax.experimental import pallas as pl
from jax.experimental.pallas import tpu as pltpu

N_DEV = 32
N_RIGHT = 16
N_LEFT = 15
S = 4

RING = [0, 8, 16, 24, 27, 28, 31, 30, 29, 26, 25, 17, 18, 21, 22, 23,
        20, 19, 11, 12, 15, 14, 13, 10, 9, 1, 2, 5, 6, 7, 4, 3]


def kernel(x, w_mat):
    m_per, k = x.shape
    _, n_per = w_mat.shape
    hm = m_per // 2

    my = lax.axis_index("i")
    ring = jnp.array(RING, dtype=jnp.int32)
    pos = jnp.argmax(ring == my).astype(jnp.int32)
    origins_r = ring[jnp.mod(pos - 1 - jnp.arange(N_RIGHT, dtype=jnp.int32), N_DEV)]
    origins_l = ring[jnp.mod(pos + 1 + jnp.arange(N_LEFT, dtype=jnp.int32), N_DEV)]
    meta = jnp.concatenate([
        ring[jnp.mod(pos - 1, N_DEV)][None],
        ring[jnp.mod(pos + 1, N_DEV)][None],
        origins_r,
        origins_l,
    ]).astype(jnp.int32)

    def body(meta_ref, x_ref, w_ref, out_ref, comm_r, comm_l,
             send_r, recv_r, send_l, recv_l, credit_r, credit_l):
        my_dev = lax.axis_index("i")
        left = meta_ref[0]
        right = meta_ref[1]

        barrier = pltpu.get_barrier_semaphore()
        for nbr in (left, right):
            pl.semaphore_signal(
                barrier, inc=1, device_id=(nbr,),
                device_id_type=pl.DeviceIdType.MESH,
            )
        pl.semaphore_wait(barrier, 2)

        def gemm_chunk(chunk, origin):
            y = jnp.dot(chunk, w_ref[...], preferred_element_type=jnp.float32)
            out_ref[pl.ds(origin * m_per, m_per), :] = jnp.maximum(y, 0.0)

        def sub(ref_slot, j):
            return ref_slot.at[pl.ds(j * hm, hm), :]

        def start_sub(comm, sends, recvs, h, j, dev):
            s = h % S
            src = x_ref if h == 0 else comm.at[(h - 1) % S]
            rdma = pltpu.make_async_remote_copy(
                src_ref=sub(src, j),
                dst_ref=sub(comm.at[s], j),
                send_sem=sends.at[s, j],
                recv_sem=recvs.at[s, j],
                device_id=(dev,),
                device_id_type=pl.DeviceIdType.MESH,
            )
            rdma.start()
            return rdma

        prev = {"r0": None, "r1": None, "l0": None, "l1": None}
        for h in range(N_RIGHT):
            if h >= S:
                pl.semaphore_wait(credit_r, 1)
            if h >= S and h < N_LEFT:
                pl.semaphore_wait(credit_l, 1)

            r0 = start_sub(comm_r, send_r, recv_r, h, 0, right)
            l0 = start_sub(comm_l, send_l, recv_l, h, 0, left) if h < N_LEFT else None

            if prev["r1"] is not None:
                prev["r1"].wait_recv()
            if prev["l1"] is not None:
                prev["l1"].wait_recv()

            r1 = start_sub(comm_r, send_r, recv_r, h, 1, right)
            l1 = start_sub(comm_l, send_l, recv_l, h, 1, left) if h < N_LEFT else None

            if prev["r0"] is not None:
                prev["r0"].wait_send()
                prev["r1"].wait_send()
                if 0 <= h - 2 <= N_RIGHT - S - 1:
                    pl.semaphore_signal(
                        credit_r, inc=1, device_id=(left,),
                        device_id_type=pl.DeviceIdType.MESH,
                    )
            if prev["l0"] is not None:
                prev["l0"].wait_send()
                prev["l1"].wait_send()
                if 0 <= h - 2 <= N_LEFT - S - 1:
                    pl.semaphore_signal(
                        credit_l, inc=1, device_id=(right,),
                        device_id_type=pl.DeviceIdType.MESH,
                    )

            if h == 0:
                gemm_chunk(x_ref[...], my_dev)
            else:
                gemm_chunk(comm_r[(h - 1) % S], meta_ref[2 + (h - 1)])
                gemm_chunk(comm_l[(h - 1) % S], meta_ref[18 + (h - 1)])

            r0.wait_recv()
            if l0 is not None:
                l0.wait_recv()

            prev["r0"], prev["r1"] = r0, r1
            if l0 is not None:
                prev["l0"], prev["l1"] = l0, l1

        prev["r1"].wait_recv()
        prev["r0"].wait_send()
        prev["r1"].wait_send()
        gemm_chunk(comm_r[(N_RIGHT - 1) % S], meta_ref[2 + (N_RIGHT - 1)])

    return pl.pallas_call(
        body,
        out_shape=jax.ShapeDtypeStruct((N_DEV * m_per, n_per), jnp.float32),
        in_specs=[
            pl.BlockSpec(memory_space=pltpu.SMEM),
            pl.BlockSpec(memory_space=pltpu.VMEM),
            pl.BlockSpec(memory_space=pltpu.VMEM),
        ],
        out_specs=pl.BlockSpec(memory_space=pltpu.VMEM),
        scratch_shapes=[
            pltpu.VMEM((S, m_per, k), jnp.float32),
            pltpu.VMEM((S, m_per, k), jnp.float32),
            pltpu.SemaphoreType.DMA((S, 2)),
            pltpu.SemaphoreType.DMA((S, 2)),
            pltpu.SemaphoreType.DMA((S, 2)),
            pltpu.SemaphoreType.DMA((S, 2)),
            pltpu.SemaphoreType.REGULAR,
            pltpu.SemaphoreType.REGULAR,
        ],
        compiler_params=pltpu.CompilerParams(collective_id=0),
    )(meta, x, w_mat)


# device time: 366238 ns/iter; 2.0819x vs baseline; 1.0233x over previous
import jax
import jax.numpy as jnp
from jax import lax
from jax.experimental import pallas as pl
from jax.experimental.pallas import tpu as pltpu

N_DEV = 32
N_RIGHT = 16
N_LEFT = 15
S = 4

RING = [0, 8, 16, 24, 27, 28, 31, 30, 29, 26, 25, 17, 18, 21, 22, 23,
        20, 19, 11, 12, 15, 14, 13, 10, 9, 1, 2, 5, 6, 7, 4, 3]


def kernel(x, w_mat):
    m_per, k = x.shape
    _, n_per = w_mat.shape
    hm = m_per // 2

    my = lax.axis_index("i")
    ring = jnp.array(RING, dtype=jnp.int32)
    pos = jnp.argmax(ring == my).astype(jnp.int32)
    origins_r = ring[jnp.mod(pos - 1 - jnp.arange(N_RIGHT, dtype=jnp.int32), N_DEV)]
    origins_l = ring[jnp.mod(pos + 1 + jnp.arange(N_LEFT, dtype=jnp.int32), N_DEV)]
    meta = jnp.concatenate([
        ring[jnp.mod(pos - 1, N_DEV)][None],
        ring[jnp.mod(pos + 1, N_DEV)][None],
        origins_r,
        origins_l,
    ]).astype(jnp.int32)

    def body(meta_ref, x_ref, w_ref, out_ref, comm_r, comm_l,
             send_r, recv_r, send_l, recv_l, credit_r, credit_l):
        my_dev = lax.axis_index("i")
        left = meta_ref[0]
        right = meta_ref[1]

        barrier = pltpu.get_barrier_semaphore()
        for nbr in (left, right):
            pl.semaphore_signal(
                barrier, inc=1, device_id=(nbr,),
                device_id_type=pl.DeviceIdType.MESH,
            )
        pl.semaphore_wait(barrier, 2)

        def gemm_chunk(chunk, origin):
            y = jnp.dot(chunk, w_ref[...], preferred_element_type=jnp.float32)
            out_ref[pl.ds(origin * m_per, m_per), :] = jnp.maximum(y, 0.0)

        def sub(ref_slot, j):
            return ref_slot.at[pl.ds(j * hm, hm), :]

        def start_sub(comm, sends, recvs, h, j, dev):
            s = h % S
            src = x_ref if h == 0 else comm.at[(h - 1) % S]
            rdma = pltpu.make_async_remote_copy(
                src_ref=sub(src, j),
                dst_ref=sub(comm.at[s], j),
                send_sem=sends.at[s, j],
                recv_sem=recvs.at[s, j],
                device_id=(dev,),
                device_id_type=pl.DeviceIdType.MESH,
            )
            rdma.start()
            return rdma

        prev = {"r0": None, "r1": None, "l0": None, "l1": None}
        for h in range(N_RIGHT):
            if h >= S:
                pl.semaphore_wait(credit_r, 1)
                pl.semaphore_wait(credit_l, 1)

            r0 = start_sub(comm_r, send_r, recv_r, h, 0, right)
            l0 = start_sub(comm_l, send_l, recv_l, h, 0, left) if h < N_LEFT else None

            if prev["r1"] is not None:
                prev["r1"].wait_recv()
            if prev["l1"] is not None:
                prev["l1"].wait_recv()

            r1 = start_sub(comm_r, send_r, recv_r, h, 1, right) if h < N_LEFT else None
            l1 = start_sub(comm_l, send_l, recv_l, h, 1, left)

            if prev["r0"] is not None:
                prev["r0"].wait_send()
                prev["r1"].wait_send()
                if 0 <= h - 2 <= N_RIGHT - S - 1:
                    pl.semaphore_signal(
                        credit_r, inc=1, device_id=(left,),
                        device_id_type=pl.DeviceIdType.MESH,
                    )
            if prev["l0"] is not None:
                prev["l0"].wait_send()
                prev["l1"].wait_send()
                if 0 <= h - 2 <= N_RIGHT - S - 1:
                    pl.semaphore_signal(
                        credit_l, inc=1, device_id=(right,),
                        device_id_type=pl.DeviceIdType.MESH,
                    )

            if h == 0:
                gemm_chunk(x_ref[...], my_dev)
            else:
                gemm_chunk(comm_r[(h - 1) % S], meta_ref[2 + (h - 1)])
                gemm_chunk(comm_l[(h - 1) % S], meta_ref[18 + (h - 1)])

            r0.wait_recv()
            if l0 is not None:
                l0.wait_recv()

            prev["r0"], prev["l1"] = r0, l1
            if r1 is not None:
                prev["r1"] = r1
            if l0 is not None:
                prev["l0"] = l0

        prev["l1"].wait_recv()
        prev["r0"].wait_send()
        prev["l1"].wait_send()
        s_last = (N_RIGHT - 1) % S
        origin16 = meta_ref[2 + (N_RIGHT - 1)]
        y0 = jnp.dot(comm_r[s_last, :hm, :], w_ref[...],
                     preferred_element_type=jnp.float32)
        out_ref[pl.ds(origin16 * m_per, hm), :] = jnp.maximum(y0, 0.0)
        y1 = jnp.dot(comm_l[s_last, hm:, :], w_ref[...],
                     preferred_element_type=jnp.float32)
        out_ref[pl.ds(origin16 * m_per + hm, hm), :] = jnp.maximum(y1, 0.0)

    return pl.pallas_call(
        body,
        out_shape=jax.ShapeDtypeStruct((N_DEV * m_per, n_per), jnp.float32),
        in_specs=[
            pl.BlockSpec(memory_space=pltpu.SMEM),
            pl.BlockSpec(memory_space=pltpu.VMEM),
            pl.BlockSpec(memory_space=pltpu.VMEM),
        ],
        out_specs=pl.BlockSpec(memory_space=pltpu.VMEM),
        scratch_shapes=[
            pltpu.VMEM((S, m_per, k), jnp.float32),
            pltpu.VMEM((S, m_per, k), jnp.float32),
            pltpu.SemaphoreType.DMA((S, 2)),
            pltpu.SemaphoreType.DMA((S, 2)),
            pltpu.SemaphoreType.DMA((S, 2)),
            pltpu.SemaphoreType.DMA((S, 2)),
            pltpu.SemaphoreType.REGULAR,
            pltpu.SemaphoreType.REGULAR,
        ],
        compiler_params=pltpu.CompilerParams(collective_id=0),
    )(meta, x, w_mat)


# device time: 364370 ns/iter; 2.0926x vs baseline; 1.0051x over previous
import jax
import jax.numpy as jnp
from jax import lax
from jax.experimental import pallas as pl
from jax.experimental.pallas import tpu as pltpu

N_DEV = 32
N_HOPS = 16
N_FULL = 15
S = 4
SUBS = 4

RING = [0, 8, 16, 24, 27, 28, 31, 30, 29, 26, 25, 17, 18, 21, 22, 23,
        20, 19, 11, 12, 15, 14, 13, 10, 9, 1, 2, 5, 6, 7, 4, 3]


def _subs_r(h):
    return list(range(SUBS)) if h < N_FULL else list(range(SUBS // 2))


def _subs_l(h):
    return list(range(SUBS)) if h < N_FULL else list(range(SUBS // 2, SUBS))


def kernel(x, w_mat):
    m_per, k = x.shape
    _, n_per = w_mat.shape
    hm = m_per // SUBS
    half = m_per // 2

    my = lax.axis_index("i")
    ring = jnp.array(RING, dtype=jnp.int32)
    pos = jnp.argmax(ring == my).astype(jnp.int32)
    origins_r = ring[jnp.mod(pos - 1 - jnp.arange(N_HOPS, dtype=jnp.int32), N_DEV)]
    origins_l = ring[jnp.mod(pos + 1 + jnp.arange(N_FULL, dtype=jnp.int32), N_DEV)]
    meta = jnp.concatenate([
        ring[jnp.mod(pos - 1, N_DEV)][None],
        ring[jnp.mod(pos + 1, N_DEV)][None],
        origins_r,
        origins_l,
    ]).astype(jnp.int32)

    def body(meta_ref, x_ref, w_ref, out_ref, comm_r, comm_l,
             send_r, recv_r, send_l, recv_l, credit_r, credit_l):
        my_dev = lax.axis_index("i")
        left = meta_ref[0]
        right = meta_ref[1]

        barrier = pltpu.get_barrier_semaphore()
        for nbr in (left, right):
            pl.semaphore_signal(
                barrier, inc=1, device_id=(nbr,),
                device_id_type=pl.DeviceIdType.MESH,
            )
        pl.semaphore_wait(barrier, 2)

        def gemm_chunk(chunk, origin):
            y = jnp.dot(chunk, w_ref[...], preferred_element_type=jnp.float32)
            out_ref[pl.ds(origin * m_per, m_per), :] = jnp.maximum(y, 0.0)

        def sub(ref_slot, j):
            return ref_slot.at[pl.ds(j * hm, hm), :]

        def start_sub(comm, sends, recvs, h, j, dev):
            s = h % S
            src = x_ref if h == 0 else comm.at[(h - 1) % S]
            rdma = pltpu.make_async_remote_copy(
                src_ref=sub(src, j),
                dst_ref=sub(comm.at[s], j),
                send_sem=sends.at[s, j],
                recv_sem=recvs.at[s, j],
                device_id=(dev,),
                device_id_type=pl.DeviceIdType.MESH,
            )
            rdma.start()
            return rdma

        prev_r = {}
        prev_l = {}
        for h in range(N_HOPS):
            subs_r, subs_l = _subs_r(h), _subs_l(h)
            if h >= S:
                pl.semaphore_wait(credit_r, 1)
                pl.semaphore_wait(credit_l, 1)

            cur_r = {}
            cur_l = {}
            for j in range(SUBS):
                if j > 0 and j in prev_r:
                    prev_r[j].wait_recv()
                if j > 0 and j in prev_l:
                    prev_l[j].wait_recv()
                if j in subs_r:
                    cur_r[j] = start_sub(comm_r, send_r, recv_r, h, j, right)
                if j in subs_l:
                    cur_l[j] = start_sub(comm_l, send_l, recv_l, h, j, left)

            if prev_r:
                for d in prev_r.values():
                    d.wait_send()
                if 0 <= h - 2 <= N_HOPS - S - 1:
                    pl.semaphore_signal(
                        credit_r, inc=1, device_id=(left,),
                        device_id_type=pl.DeviceIdType.MESH,
                    )
            if prev_l:
                for d in prev_l.values():
                    d.wait_send()
                if 0 <= h - 2 <= N_HOPS - S - 1:
                    pl.semaphore_signal(
                        credit_l, inc=1, device_id=(right,),
                        device_id_type=pl.DeviceIdType.MESH,
                    )

            if h == 0:
                gemm_chunk(x_ref[...], my_dev)
            else:
                gemm_chunk(comm_r[(h - 1) % S], meta_ref[2 + (h - 1)])
                gemm_chunk(comm_l[(h - 1) % S], meta_ref[18 + (h - 1)])

            if 0 in cur_r:
                cur_r[0].wait_recv()
            if 0 in cur_l:
                cur_l[0].wait_recv()

            prev_r, prev_l = cur_r, cur_l

        for j, d in prev_r.items():
            if j != 0:
                d.wait_recv()
            d.wait_send()
        for d in prev_l.values():
            d.wait_recv()
            d.wait_send()
        s_last = (N_HOPS - 1) % S
        origin16 = meta_ref[2 + (N_HOPS - 1)]
        y0 = jnp.dot(comm_r[s_last, :half, :], w_ref[...],
                     preferred_element_type=jnp.float32)
        out_ref[pl.ds(origin16 * m_per, half), :] = jnp.maximum(y0, 0.0)
        y1 = jnp.dot(comm_l[s_last, half:, :], w_ref[...],
                     preferred_element_type=jnp.float32)
        out_ref[pl.ds(origin16 * m_per + half, half), :] = jnp.maximum(y1, 0.0)

    return pl.pallas_call(
        body,
        out_shape=jax.ShapeDtypeStruct((N_DEV * m_per, n_per), jnp.float32),
        in_specs=[
            pl.BlockSpec(memory_space=pltpu.SMEM),
            pl.BlockSpec(memory_space=pltpu.VMEM),
            pl.BlockSpec(memory_space=pltpu.VMEM),
        ],
        out_specs=pl.BlockSpec(memory_space=pltpu.VMEM),
        scratch_shapes=[
            pltpu.VMEM((S, m_per, k), jnp.float32),
            pltpu.VMEM((S, m_per, k), jnp.float32),
            pltpu.SemaphoreType.DMA((S, SUBS)),
            pltpu.SemaphoreType.DMA((S, SUBS)),
            pltpu.SemaphoreType.DMA((S, SUBS)),
            pltpu.SemaphoreType.DMA((S, SUBS)),
            pltpu.SemaphoreType.REGULAR,
            pltpu.SemaphoreType.REGULAR,
        ],
        compiler_params=pltpu.CompilerParams(collective_id=0),
    )(meta, x, w_mat)
